# baseline (speedup 1.0000x reference)
import functools

import jax
import jax.numpy as jnp
from jax import lax
from jax.experimental import pallas as pl
from jax.experimental.pallas import tpu as pltpu
from jax.experimental.pallas import tpu_sc as plsc

D = 64
B = 4096
HIST = 50
NC, NS, L = 2, 16, 16
NW = NC * NS
BPW = B // NW

_mesh = plsc.VectorSubcoreMesh(core_axis_name="c", subcore_axis_name="s")


@functools.partial(
    pl.kernel,
    mesh=_mesh,
    out_type=jax.ShapeDtypeStruct((B,), jnp.float32),
    scratch_types=(
        [
            pltpu.VMEM((HIST, BPW), jnp.int32),
            pltpu.VMEM((BPW,), jnp.int32),
            pltpu.VMEM((BPW, D), jnp.float32),
            pltpu.VMEM((BPW, D), jnp.float32),
            pltpu.VMEM((BPW,), jnp.float32),
            pltpu.VMEM((BPW,), jnp.float32),
            pltpu.VMEM((L,), jnp.int32),
            pltpu.VMEM((L, D), jnp.float32),
        ]
        + [pltpu.SemaphoreType.DMA, pltpu.SemaphoreType.DMA,
           pltpu.SemaphoreType.DMA]
    ),
    compiler_params=pltpu.CompilerParams(use_tc_tiling_on_sc=False),
)
def _bprmf_sc(seq_hbm, tgt_hbm, table_hbm, out_hbm,
              st, tgt_idx, tgt_rows, acc, wbuf, out_buf, pidx, prows,
              gsem, tsem, psem):
    wid = lax.axis_index("s") * NC + lax.axis_index("c")
    base = wid * BPW

    lane = lax.iota(jnp.int32, L)
    zero = jnp.zeros((L,), jnp.float32)
    one = jnp.ones((L,), jnp.float32)

    pltpu.sync_copy(seq_hbm.at[:, pl.ds(base, BPW)], st)
    pltpu.sync_copy(tgt_hbm.at[wid], tgt_idx)

    pltpu.async_copy(table_hbm.at[tgt_idx], tgt_rows, tsem)

    pad_vec = jnp.full((L,), 1, jnp.int32) + wid
    pidx[pl.ds(0, L)] = pad_vec
    pltpu.async_copy(table_hbm.at[pidx], prows, psem)

    def _zbody(b, carry):
        for k in range(D // L):
            acc[b, pl.ds(k * L, L)] = zero
        return carry
    lax.fori_loop(0, BPW, _zbody, 0, unroll=8)

    def _cbody(t, carry):
        w = zero
        for l in range(HIST):
            s = st[l, pl.ds(t * L, L)]
            w = w + jnp.where(s != 0, one, zero)
            st[l, pl.ds(t * L, L)] = jnp.where(s == 0, pad_vec, s)
        wbuf[pl.ds(t * L, L)] = w
        return carry
    lax.fori_loop(0, BPW // L, _cbody, 0)

    descs = [pltpu.async_copy(table_hbm.at[st.at[l]], acc, gsem, add=True)
             for l in range(HIST)]

    pltpu.make_async_copy(table_hbm.at[tgt_idx], tgt_rows, tsem).wait()
    pltpu.make_async_copy(table_hbm.at[pidx], prows, psem).wait()
    for d in descs:
        d.wait()

    def _allreduce_sum(v):
        for k in (8, 4, 2, 1):
            v = v + v.at[lane ^ k].get(mode="promise_in_bounds")
        return v

    def _obody(t, carry):
        res = zero
        w = wbuf[pl.ds(t * L, L)]
        npad = jnp.full((L,), float(HIST), jnp.float32) - w
        for j in range(L):
            b = t * L + j
            nb = npad.at[jnp.full((L,), j, jnp.int32)].get(
                mode="promise_in_bounds")
            dotv = zero
            for k in range(D // L):
                dotv = dotv + (
                    (acc[b, pl.ds(k * L, L)] - nb * prows[0, pl.ds(k * L, L)])
                    * tgt_rows[b, pl.ds(k * L, L)])
            pred_v = _allreduce_sum(dotv)
            res = jnp.where(lane == j, pred_v, res)
        out_buf[pl.ds(t * L, L)] = res / w
        return carry
    lax.fori_loop(0, BPW // L, _obody, 0)

    pltpu.sync_copy(out_buf, out_hbm.at[pl.ds(base, BPW)])


def kernel(seq, target, embed_weight):
    seq_t = jnp.swapaxes(seq.astype(jnp.int32), 0, 1)
    tgt_w = target.astype(jnp.int32).reshape(NW, BPW)
    return _bprmf_sc(seq_t, tgt_w, embed_weight)
